# baseline (speedup 1.0000x reference)
import functools

import jax
import jax.numpy as jnp
from jax import lax
from jax.experimental import pallas as pl
from jax.experimental.pallas import tpu as pltpu
from jax.experimental.pallas import tpu_sc as plsc

V, D, B, T = 8192, 1024, 8, 3072
H, W = 32, 32

_info = plsc.get_sparse_core_info()
NC, NS, L = _info.num_cores, _info.num_subcores, _info.num_lanes
NW = NC * NS
BT = B * T
PER_W = BT // NW
BLK = W
NBLK = PER_W // BLK
DV = D // L


def _sc_body(idx_hbm, tok_hbm, row_hbm, col_hbm, chan_hbm, out_hbm,
             idx_v, col_v, tok_v, rc_v, row_v, chan_v, sem):
    wid = lax.axis_index("s") * NC + lax.axis_index("c")
    base = wid * PER_W
    t0 = base % T

    pltpu.sync_copy(idx_hbm.at[pl.ds(base, PER_W)], idx_v)
    pltpu.sync_copy(col_hbm, col_v)

    def blk_body(blk, _):
        t_blk = t0 + blk * BLK
        r = (t_blk % (H * W)) // W
        ch = t_blk // (H * W)
        pltpu.sync_copy(row_hbm.at[r], row_v)
        pltpu.sync_copy(chan_hbm.at[ch], chan_v)

        cp = pltpu.async_copy(
            tok_hbm.at[idx_v.at[pl.ds(blk * BLK, BLK)]], tok_v, sem)

        def rc_body(i, _):
            sl = pl.ds(i * L, L)
            rc_v[sl] = row_v[sl] + chan_v[sl]
            return 0
        lax.fori_loop(0, DV, rc_body, 0)

        cp.wait()

        def add_i(i, _):
            sl = pl.ds(i * L, L)
            rc16 = rc_v[sl]

            def add_j(j, _):
                tok_v[j, sl] = tok_v[j, sl] + col_v[j, sl] + rc16
                return 0
            lax.fori_loop(0, BLK, add_j, 0)
            return 0
        lax.fori_loop(0, DV, add_i, 0)

        pltpu.sync_copy(tok_v, out_hbm.at[pl.ds(base + blk * BLK, BLK)])
        return 0

    lax.fori_loop(0, NBLK, blk_body, 0)


@jax.jit
def _run(idx_flat, tok_table, row_table, col_table, chan_table):
    mesh = plsc.VectorSubcoreMesh(core_axis_name="c", subcore_axis_name="s")
    k = functools.partial(
        pl.kernel, mesh=mesh,
        out_type=jax.ShapeDtypeStruct((BT, D), jnp.float32),
        scratch_types=[
            pltpu.VMEM((PER_W,), jnp.int32),
            pltpu.VMEM((BLK, D), jnp.float32),
            pltpu.VMEM((BLK, D), jnp.float32),
            pltpu.VMEM((D,), jnp.float32),
            pltpu.VMEM((D,), jnp.float32),
            pltpu.VMEM((D,), jnp.float32),
            pltpu.SemaphoreType.DMA,
        ],
    )(_sc_body)
    return k(idx_flat, tok_table, row_table, col_table, chan_table)


def kernel(idx, tok_table, row_table, col_table, chan_table):
    idx_flat = idx.reshape(-1).astype(jnp.int32)
    out = _run(idx_flat, tok_table, row_table, col_table, chan_table)
    return out.reshape(B, T, D)


# speedup vs baseline: 1.4568x; 1.4568x over previous
import functools

import jax
import jax.numpy as jnp
from jax import lax
from jax.experimental import pallas as pl
from jax.experimental.pallas import tpu as pltpu
from jax.experimental.pallas import tpu_sc as plsc

V, D, B, T = 8192, 1024, 8, 3072
H, W = 32, 32

_info = plsc.get_sparse_core_info()
NC, NS, L = _info.num_cores, _info.num_subcores, _info.num_lanes
NW = NC * NS
BT = B * T
BLK = W
NTB = T // BLK
TB_PER_W = NTB // NW
UNITS = TB_PER_W * B
DV = D // L
UNROLL = 16


def _sc_body(idx_hbm, tok_hbm, row_hbm, col_hbm, chan_hbm, out_hbm,
             idx_v, pos_v, tok_a, tok_b, row_v, chan_v,
             gsem_a, gsem_b, ssem_a, ssem_b):
    wid = lax.axis_index("s") * NC + lax.axis_index("c")
    base = wid * (UNITS * BLK)

    pltpu.sync_copy(idx_hbm.at[pl.ds(base, UNITS * BLK)], idx_v)

    bufs = (tok_a, tok_b)
    gsems = (gsem_a, gsem_b)
    ssems = (ssem_a, ssem_b)

    def gather(u):
        return pltpu.async_copy(
            tok_hbm.at[idx_v.at[pl.ds(u * BLK, BLK)]],
            bufs[u % 2], gsems[u % 2])

    def build_posblk(k):
        tpos = (wid * TB_PER_W + k) * BLK
        r = (tpos % (H * W)) // W
        ch = tpos // (H * W)
        pltpu.sync_copy(row_hbm.at[r], row_v)
        pltpu.sync_copy(chan_hbm.at[ch], chan_v)
        pltpu.sync_copy(col_hbm, pos_v)

        def rc_body(i, _):
            sl = pl.ds(i * L, L)
            rc16 = row_v[sl] + chan_v[sl]

            def rc_j(j, _):
                plsc.addupdate(pos_v.at[j, sl], rc16)
                return 0
            lax.fori_loop(0, BLK, rc_j, 0)
            return 0
        lax.fori_loop(0, DV, rc_body, 0)

    def add_pos(buf):
        def add_j(j, _):
            def add_i(i8, _):
                for s in range(UNROLL):
                    sl = pl.ds(i8 * (UNROLL * L) + s * L, L)
                    plsc.addupdate(buf.at[j, sl], pos_v[j, sl])
                return 0
            lax.fori_loop(0, DV // UNROLL, add_i, 0)
            return 0
        lax.fori_loop(0, BLK, add_j, 0)

    gather_cp = {0: gather(0)}
    scatter_cp = {}
    for u in range(UNITS):
        k, b = divmod(u, B)
        if b == 0:
            build_posblk(k)
        gather_cp[u].wait()
        if u + 1 < UNITS:
            if u >= 1:
                scatter_cp[u - 1].wait()
            gather_cp[u + 1] = gather(u + 1)
        add_pos(bufs[u % 2])
        dst = b * T + (wid * TB_PER_W + k) * BLK
        scatter_cp[u] = pltpu.async_copy(
            bufs[u % 2], out_hbm.at[pl.ds(dst, BLK)], ssems[u % 2])
    scatter_cp[UNITS - 2].wait()
    scatter_cp[UNITS - 1].wait()


@jax.jit
def _run(idx_r, tok_table, row_table, col_table, chan_table):
    mesh = plsc.VectorSubcoreMesh(core_axis_name="c", subcore_axis_name="s")
    k = functools.partial(
        pl.kernel, mesh=mesh,
        out_type=jax.ShapeDtypeStruct((BT, D), jnp.float32),
        scratch_types=[
            pltpu.VMEM((UNITS * BLK,), jnp.int32),
            pltpu.VMEM((BLK, D), jnp.float32),
            pltpu.VMEM((BLK, D), jnp.float32),
            pltpu.VMEM((BLK, D), jnp.float32),
            pltpu.VMEM((D,), jnp.float32),
            pltpu.VMEM((D,), jnp.float32),
            pltpu.SemaphoreType.DMA,
            pltpu.SemaphoreType.DMA,
            pltpu.SemaphoreType.DMA,
            pltpu.SemaphoreType.DMA,
        ],
    )(_sc_body)
    return k(idx_r, tok_table, row_table, col_table, chan_table)


def kernel(idx, tok_table, row_table, col_table, chan_table):
    idx_r = (idx.astype(jnp.int32)
             .reshape(B, NTB, BLK).transpose(1, 0, 2).reshape(-1))
    out = _run(idx_r, tok_table, row_table, col_table, chan_table)
    return out.reshape(B, T, D)


# speedup vs baseline: 3.1089x; 2.1340x over previous
import functools

import jax
import jax.numpy as jnp
from jax import lax
from jax.experimental import pallas as pl
from jax.experimental.pallas import tpu as pltpu
from jax.experimental.pallas import tpu_sc as plsc

V, D, B, T = 8192, 1024, 8, 3072
H, W = 32, 32

_info = plsc.get_sparse_core_info()
NC, NS, L = _info.num_cores, _info.num_subcores, _info.num_lanes
NW = NC * NS
BT = B * T
BLK = W
NTB = T // BLK
TB_PER_W = NTB // NW
UNITS = TB_PER_W * B
DV = D // L
UNROLL = 8


def _sc_body(idx_hbm, tok_hbm, row_hbm, col_hbm, chan_hbm, out_hbm,
             idx_v, pos_v, tok_a, tok_b, row_v, chan_v,
             gsem_a, gsem_b, ssem_a, ssem_b):
    wid = lax.axis_index("s") * NC + lax.axis_index("c")
    base = wid * (UNITS * BLK)

    pltpu.sync_copy(idx_hbm.at[pl.ds(base, UNITS * BLK)], idx_v)

    bufs = (tok_a, tok_b)
    gsems = (gsem_a, gsem_b)
    ssems = (ssem_a, ssem_b)

    def gather(u):
        return pltpu.async_copy(
            tok_hbm.at[idx_v.at[pl.ds(u * BLK, BLK)]],
            bufs[u % 2], gsems[u % 2])

    def build_posblk(k):
        tpos = (wid * TB_PER_W + k) * BLK
        r = (tpos % (H * W)) // W
        ch = tpos // (H * W)
        pltpu.sync_copy(row_hbm.at[r], row_v)
        pltpu.sync_copy(chan_hbm.at[ch], chan_v)
        pltpu.sync_copy(col_hbm, pos_v)

        def rc_body(i, _):
            sl = pl.ds(i * L, L)
            rc16 = row_v[sl] + chan_v[sl]

            @plsc.parallel_loop(0, BLK, unroll=8)
            def rc_j(j):
                plsc.addupdate(pos_v.at[j, sl], rc16)
            return 0
        lax.fori_loop(0, DV, rc_body, 0)

    def add_pos(buf):
        def add_j(j, _):
            @plsc.parallel_loop(0, DV, unroll=UNROLL)
            def add_i(i):
                sl = pl.ds(i * L, L)
                plsc.addupdate(buf.at[j, sl], pos_v[j, sl])
            return 0
        lax.fori_loop(0, BLK, add_j, 0)

    gather_cp = {0: gather(0)}
    scatter_cp = {}
    for u in range(UNITS):
        k, b = divmod(u, B)
        if b == 0:
            build_posblk(k)
        gather_cp[u].wait()
        if u + 1 < UNITS:
            if u >= 1:
                scatter_cp[u - 1].wait()
            gather_cp[u + 1] = gather(u + 1)
        add_pos(bufs[u % 2])
        dst = b * T + (wid * TB_PER_W + k) * BLK
        scatter_cp[u] = pltpu.async_copy(
            bufs[u % 2], out_hbm.at[pl.ds(dst, BLK)], ssems[u % 2])
    scatter_cp[UNITS - 2].wait()
    scatter_cp[UNITS - 1].wait()


@jax.jit
def _run(idx_r, tok_table, row_table, col_table, chan_table):
    mesh = plsc.VectorSubcoreMesh(core_axis_name="c", subcore_axis_name="s")
    k = functools.partial(
        pl.kernel, mesh=mesh,
        out_type=jax.ShapeDtypeStruct((BT, D), jnp.float32),
        scratch_types=[
            pltpu.VMEM((UNITS * BLK,), jnp.int32),
            pltpu.VMEM((BLK, D), jnp.float32),
            pltpu.VMEM((BLK, D), jnp.float32),
            pltpu.VMEM((BLK, D), jnp.float32),
            pltpu.VMEM((D,), jnp.float32),
            pltpu.VMEM((D,), jnp.float32),
            pltpu.SemaphoreType.DMA,
            pltpu.SemaphoreType.DMA,
            pltpu.SemaphoreType.DMA,
            pltpu.SemaphoreType.DMA,
        ],
    )(_sc_body)
    return k(idx_r, tok_table, row_table, col_table, chan_table)


def kernel(idx, tok_table, row_table, col_table, chan_table):
    idx_r = (idx.astype(jnp.int32)
             .reshape(B, NTB, BLK).transpose(1, 0, 2).reshape(-1))
    out = _run(idx_r, tok_table, row_table, col_table, chan_table)
    return out.reshape(B, T, D)


# speedup vs baseline: 3.6148x; 1.1627x over previous
import functools

import jax
import jax.numpy as jnp
from jax import lax
from jax.experimental import pallas as pl
from jax.experimental.pallas import tpu as pltpu
from jax.experimental.pallas import tpu_sc as plsc

V, D, B, T = 8192, 1024, 8, 3072
H, W = 32, 32

_info = plsc.get_sparse_core_info()
NC, NS, L = _info.num_cores, _info.num_subcores, _info.num_lanes
NW = NC * NS
BT = B * T
BLK = W
NTB = T // BLK
TB_PER_W = NTB // NW
UNITS = TB_PER_W * B
DV = D // L
UNROLL = 8
HPB = 2
HROWS = BLK // HPB
HBLK = HROWS
HUNITS = UNITS * HPB
NBUF = 4
AHEAD = NBUF - 2


def _sc_body(idx_hbm, tok_hbm, row_hbm, col_hbm, chan_hbm, out_hbm,
             idx_v, pos_v, t0, t1, t2, t3, g0, g1, g2, g3,
             s0, s1, s2, s3, row_v, chan_v):
    tok_bufs = (t0, t1, t2, t3)
    gsems = (g0, g1, g2, g3)
    ssems = (s0, s1, s2, s3)
    wid = lax.axis_index("s") * NC + lax.axis_index("c")
    base = wid * (UNITS * BLK)

    pltpu.sync_copy(idx_hbm.at[pl.ds(base, UNITS * BLK)], idx_v)

    def gather(u):
        return pltpu.async_copy(
            tok_hbm.at[idx_v.at[pl.ds(u * HBLK, HBLK)]],
            tok_bufs[u % NBUF], gsems[u % NBUF])

    def build_posblk(k):
        tpos = (wid * TB_PER_W + k) * BLK
        r = (tpos % (H * W)) // W
        ch = tpos // (H * W)
        pltpu.sync_copy(row_hbm.at[r], row_v)
        pltpu.sync_copy(chan_hbm.at[ch], chan_v)
        pltpu.sync_copy(col_hbm, pos_v)

        def rc_body(i, _):
            sl = pl.ds(i * L, L)
            rc16 = row_v[sl] + chan_v[sl]

            @plsc.parallel_loop(0, BLK, unroll=8)
            def rc_j(j):
                plsc.addupdate(pos_v.at[j, sl], rc16)
            return 0
        lax.fori_loop(0, DV, rc_body, 0)

    def add_pos(buf, h):
        def add_j(j, _):
            @plsc.parallel_loop(0, DV, unroll=UNROLL)
            def add_i(i):
                sl = pl.ds(i * L, L)
                plsc.addupdate(buf.at[j, sl], pos_v[h * HROWS + j, sl])
            return 0
        lax.fori_loop(0, HROWS, add_j, 0)

    gather_cp = {}
    scatter_cp = {}
    for u in range(AHEAD):
        gather_cp[u] = gather(u)
    for u in range(HUNITS):
        k, bh = divmod(u, B * HPB)
        b, h = divmod(bh, HPB)
        if bh == 0:
            build_posblk(k)
        gather_cp[u].wait()
        nxt = u + AHEAD
        if nxt < HUNITS:
            if nxt - NBUF >= 0:
                scatter_cp[nxt - NBUF].wait()
            gather_cp[nxt] = gather(nxt)
        add_pos(tok_bufs[u % NBUF], h)
        dst = b * T + (wid * TB_PER_W + k) * BLK + h * HROWS
        scatter_cp[u] = pltpu.async_copy(
            tok_bufs[u % NBUF], out_hbm.at[pl.ds(dst, HROWS)], ssems[u % NBUF])
    for u in range(HUNITS - NBUF, HUNITS):
        scatter_cp[u].wait()


@jax.jit
def _run(idx_r, tok_table, row_table, col_table, chan_table):
    mesh = plsc.VectorSubcoreMesh(core_axis_name="c", subcore_axis_name="s")
    k = functools.partial(
        pl.kernel, mesh=mesh,
        out_type=jax.ShapeDtypeStruct((BT, D), jnp.float32),
        scratch_types=(
            [pltpu.VMEM((UNITS * BLK,), jnp.int32),
             pltpu.VMEM((BLK, D), jnp.float32)]
            + [pltpu.VMEM((HROWS, D), jnp.float32)] * NBUF
            + [pltpu.SemaphoreType.DMA] * (2 * NBUF)
            + [pltpu.VMEM((D,), jnp.float32),
               pltpu.VMEM((D,), jnp.float32)]
        ),
    )(_sc_body)
    return k(idx_r, tok_table, row_table, col_table, chan_table)


def kernel(idx, tok_table, row_table, col_table, chan_table):
    idx_r = (idx.astype(jnp.int32)
             .reshape(B, NTB, BLK).transpose(1, 0, 2).reshape(-1))
    out = _run(idx_r, tok_table, row_table, col_table, chan_table)
    return out.reshape(B, T, D)


# speedup vs baseline: 3.6215x; 1.0019x over previous
import functools

import jax
import jax.numpy as jnp
from jax import lax
from jax.experimental import pallas as pl
from jax.experimental.pallas import tpu as pltpu
from jax.experimental.pallas import tpu_sc as plsc

V, D, B, T = 8192, 1024, 8, 3072
H, W = 32, 32

_info = plsc.get_sparse_core_info()
NC, NS, L = _info.num_cores, _info.num_subcores, _info.num_lanes
NW = NC * NS
BT = B * T
BLK = W
NTB = T // BLK
TB_PER_W = NTB // NW
UNITS = TB_PER_W * B
DV = D // L
UNROLL = 8
HPB = 2
HROWS = BLK // HPB
HBLK = HROWS
HUNITS = UNITS * HPB
NBUF = 5
AHEAD = NBUF - 2


def _sc_body(idx_hbm, tok_hbm, row_hbm, col_hbm, chan_hbm, out_hbm,
             idx_v, pos_v, t0, t1, t2, t3, t4, g0, g1, g2, g3, g4,
             s0, s1, s2, s3, s4, isem, row_v, chan_v):
    tok_bufs = (t0, t1, t2, t3, t4)
    gsems = (g0, g1, g2, g3, g4)
    ssems = (s0, s1, s2, s3, s4)
    wid = lax.axis_index("s") * NC + lax.axis_index("c")

    idx_cps = []
    for k in range(TB_PER_W):
        for b in range(B):
            src = b * T + (wid * TB_PER_W + k) * BLK
            idx_cps.append(pltpu.async_copy(
                idx_hbm.at[pl.ds(src, BLK)],
                idx_v.at[pl.ds((k * B + b) * BLK, BLK)], isem))
    for cp in idx_cps:
        cp.wait()

    def gather(u):
        return pltpu.async_copy(
            tok_hbm.at[idx_v.at[pl.ds(u * HBLK, HBLK)]],
            tok_bufs[u % NBUF], gsems[u % NBUF])

    def build_posblk(k):
        tpos = (wid * TB_PER_W + k) * BLK
        r = (tpos % (H * W)) // W
        ch = tpos // (H * W)
        pltpu.sync_copy(row_hbm.at[r], row_v)
        pltpu.sync_copy(chan_hbm.at[ch], chan_v)
        pltpu.sync_copy(col_hbm, pos_v)

        def rc_body(i, _):
            sl = pl.ds(i * L, L)
            rc16 = row_v[sl] + chan_v[sl]

            @plsc.parallel_loop(0, BLK, unroll=8)
            def rc_j(j):
                plsc.addupdate(pos_v.at[j, sl], rc16)
            return 0
        lax.fori_loop(0, DV, rc_body, 0)

    def add_pos(buf, h):
        def add_j(j, _):
            @plsc.parallel_loop(0, DV, unroll=UNROLL)
            def add_i(i):
                sl = pl.ds(i * L, L)
                plsc.addupdate(buf.at[j, sl], pos_v[h * HROWS + j, sl])
            return 0
        lax.fori_loop(0, HROWS, add_j, 0)

    gather_cp = {}
    scatter_cp = {}
    for u in range(AHEAD):
        gather_cp[u] = gather(u)
    for u in range(HUNITS):
        k, bh = divmod(u, B * HPB)
        b, h = divmod(bh, HPB)
        if bh == 0:
            build_posblk(k)
        gather_cp[u].wait()
        nxt = u + AHEAD
        if nxt < HUNITS:
            if nxt - NBUF >= 0:
                scatter_cp[nxt - NBUF].wait()
            gather_cp[nxt] = gather(nxt)
        add_pos(tok_bufs[u % NBUF], h)
        dst = b * T + (wid * TB_PER_W + k) * BLK + h * HROWS
        scatter_cp[u] = pltpu.async_copy(
            tok_bufs[u % NBUF], out_hbm.at[pl.ds(dst, HROWS)], ssems[u % NBUF])
    for u in range(HUNITS - NBUF, HUNITS):
        scatter_cp[u].wait()


@jax.jit
def _run(idx_r, tok_table, row_table, col_table, chan_table):
    mesh = plsc.VectorSubcoreMesh(core_axis_name="c", subcore_axis_name="s")
    k = functools.partial(
        pl.kernel, mesh=mesh,
        out_type=jax.ShapeDtypeStruct((BT, D), jnp.float32),
        scratch_types=(
            [pltpu.VMEM((UNITS * BLK,), jnp.int32),
             pltpu.VMEM((BLK, D), jnp.float32)]
            + [pltpu.VMEM((HROWS, D), jnp.float32)] * NBUF
            + [pltpu.SemaphoreType.DMA] * (2 * NBUF + 1)
            + [pltpu.VMEM((D,), jnp.float32),
               pltpu.VMEM((D,), jnp.float32)]
        ),
    )(_sc_body)
    return k(idx_r, tok_table, row_table, col_table, chan_table)


def kernel(idx, tok_table, row_table, col_table, chan_table):
    idx_flat = idx.astype(jnp.int32).reshape(-1)
    out = _run(idx_flat, tok_table, row_table, col_table, chan_table)
    return out.reshape(B, T, D)
